# baseline (speedup 1.0000x reference)
import functools

import jax
import jax.numpy as jnp
from jax import lax
from jax.experimental import pallas as pl
from jax.experimental.pallas import tpu as pltpu
from jax.experimental.pallas import tpu_sc as plsc

DIM = 64
NCORES = 2
NSUB = 16
NW = NCORES * NSUB
CHUNK = 128
MARGIN = 1.0
C = 0.01


@functools.partial(jax.jit, static_argnums=(0,))
def _sc_distances(bcat, ent, rel, hh, tt, rr):
    per_w = bcat // NW
    n_chunks = per_w // CHUNK
    mesh = plsc.VectorSubcoreMesh(core_axis_name="c", subcore_axis_name="s")

    @functools.partial(
        pl.kernel,
        mesh=mesh,
        compiler_params=pltpu.CompilerParams(use_tc_tiling_on_sc=False),
        out_type=[
            jax.ShapeDtypeStruct((bcat,), jnp.float32),
            jax.ShapeDtypeStruct((NW, 16), jnp.float32),
            jax.ShapeDtypeStruct((NW, 16), jnp.float32),
        ],
        scratch_types=[
            pltpu.VMEM((CHUNK,), jnp.int32),
            pltpu.VMEM((CHUNK,), jnp.int32),
            pltpu.VMEM((CHUNK,), jnp.int32),
            pltpu.VMEM((CHUNK, DIM), jnp.float32),
            pltpu.VMEM((CHUNK, DIM), jnp.float32),
            pltpu.VMEM((CHUNK, DIM), jnp.float32),
            pltpu.VMEM((CHUNK,), jnp.float32),
            pltpu.VMEM((16,), jnp.float32),
            pltpu.SemaphoreType.DMA,
        ],
    )
    def k(ent_hbm, rel_hbm, hh_hbm, tt_hbm, rr_hbm, dsq_hbm, epen_hbm,
          rpen_hbm, h_v, t_v, r_v, hrow, trow, rrow, dbuf, penbuf, sem):
        wid = lax.axis_index("s") * NCORES + lax.axis_index("c")
        base_w = wid * per_w
        lanes = lax.iota(jnp.int32, 16)
        first = lanes == 0

        dnums = lax.GatherDimensionNumbers(
            offset_dims=(), collapsed_slice_dims=(0,), start_index_map=(0,))

        def shuf(x, idx):
            return lax.gather(
                x, idx[:, None], dimension_numbers=dnums, slice_sizes=(1,),
                mode=lax.GatherScatterMode.PROMISE_IN_BOUNDS)

        def xsum(x):
            for s in (8, 4, 2, 1):
                x = x + shuf(x, lanes ^ s)
            return x

        def chunk_body(ci, accs):
            ent_acc, rel_acc = accs
            base = base_w + ci * CHUNK
            pltpu.sync_copy(hh_hbm.at[pl.ds(base, CHUNK)], h_v)
            pltpu.sync_copy(tt_hbm.at[pl.ds(base, CHUNK)], t_v)
            pltpu.sync_copy(rr_hbm.at[pl.ds(base, CHUNK)], r_v)
            c1 = pltpu.async_copy(ent_hbm.at[h_v], hrow, sem)
            c2 = pltpu.async_copy(ent_hbm.at[t_v], trow, sem)
            c3 = pltpu.async_copy(rel_hbm.at[r_v], rrow, sem)
            c1.wait()
            c2.wait()
            c3.wait()

            def group_body(g, carry):
                ea, ra = carry
                acc_d = jnp.zeros((16,), jnp.float32)
                for jj in range(16):
                    j = g * 16 + jj
                    sd = sh = st = sr = None
                    for q in range(DIM // 16):
                        hq = hrow[j, pl.ds(q * 16, 16)]
                        rq = rrow[j, pl.ds(q * 16, 16)]
                        tq = trow[j, pl.ds(q * 16, 16)]
                        d = hq + rq - tq
                        if q == 0:
                            sd, sh, st, sr = d * d, hq * hq, tq * tq, rq * rq
                        else:
                            sd = sd + d * d
                            sh = sh + hq * hq
                            st = st + tq * tq
                            sr = sr + rq * rq
                    csd = xsum(sd)
                    csh = xsum(sh)
                    cst = xsum(st)
                    csr = xsum(sr)
                    acc_d = jnp.where(lanes == jj, csd, acc_d)
                    ea = ea + jnp.where(
                        first,
                        jnp.maximum(csh - 1.0, 0.0)
                        + jnp.maximum(cst - 1.0, 0.0),
                        0.0)
                    ra = ra + jnp.where(
                        first, jnp.maximum(csr - 1.0, 0.0), 0.0)
                dbuf[pl.ds(g * 16, 16)] = acc_d
                return (ea, ra)

            accs = lax.fori_loop(
                0, CHUNK // 16, group_body, (ent_acc, rel_acc))
            pltpu.sync_copy(dbuf, dsq_hbm.at[pl.ds(base, CHUNK)])
            return accs

        zero = jnp.zeros((16,), jnp.float32)
        ent_acc, rel_acc = lax.fori_loop(0, n_chunks, chunk_body, (zero, zero))
        penbuf[...] = ent_acc
        pltpu.sync_copy(penbuf, epen_hbm.at[wid])
        penbuf[...] = rel_acc
        pltpu.sync_copy(penbuf, rpen_hbm.at[wid])

    return k(ent, rel, hh, tt, rr)


def _finalize(pos_sq, neg_sq, epen, rpen):
    batch = pos_sq.shape[0] * pos_sq.shape[1]

    def body(pos_ref, neg_ref, epen_ref, rpen_ref, out_ref):
        p = jnp.sqrt(pos_ref[...])
        n = jnp.sqrt(neg_ref[...])
        loss = jnp.sum(jnp.maximum(p - n + MARGIN, 0.0)) / batch
        ent = jnp.sum(epen_ref[...]) / (4.0 * batch)
        rel = jnp.sum(rpen_ref[...]) / (2.0 * batch)
        out_ref[...] = jnp.full((1, 1), loss + C * (ent + rel), jnp.float32)

    return pl.pallas_call(
        body,
        out_shape=jax.ShapeDtypeStruct((1, 1), jnp.float32),
    )(pos_sq, neg_sq, epen, rpen)


def kernel(triple, corrupted_triple, entity_emb, relation_emb):
    h = triple[:, 0].astype(jnp.int32)
    r = triple[:, 1].astype(jnp.int32)
    t = triple[:, 2].astype(jnp.int32)
    hc = corrupted_triple[:, 0].astype(jnp.int32)
    rc = corrupted_triple[:, 1].astype(jnp.int32)
    tc = corrupted_triple[:, 2].astype(jnp.int32)
    batch = h.shape[0]
    hh = jnp.concatenate([h, hc])
    tt = jnp.concatenate([t, tc])
    rr = jnp.concatenate([r, rc])
    dsq, epen, rpen = _sc_distances(
        2 * batch, entity_emb, relation_emb, hh, tt, rr)
    pos_sq = dsq[:batch].reshape(128, -1)
    neg_sq = dsq[batch:].reshape(128, -1)
    out = _finalize(pos_sq, neg_sq, epen, rpen)
    return out[0, 0]
